# speedup vs baseline: 1.5106x; 1.5106x over previous
import functools

import jax
import jax.numpy as jnp
from jax.experimental import pallas as pl
from jax.experimental.pallas import tpu as pltpu

_TOPK = 2
_E = 8
_BLK = 256
_NB = 4096 // _BLK + _E
_PADT = _NB * _BLK


def _glu_body(be_ref, bv_ref, xs_ref, wg_ref, wu_ref, wd_ref, ys_ref):
    b = pl.program_id(0)

    @pl.when(bv_ref[b] == 1)
    def _():
        xb = xs_ref[...]
        h = jnp.dot(xb, wg_ref[0], preferred_element_type=jnp.float32)
        u = jnp.dot(xb, wu_ref[0], preferred_element_type=jnp.float32)
        act = h * jax.lax.logistic(h) * u
        ys_ref[...] = jnp.dot(act, wd_ref[0], preferred_element_type=jnp.float32)

    @pl.when(bv_ref[b] == 0)
    def _():
        ys_ref[...] = jnp.zeros_like(ys_ref)


def kernel(x, router_w, e_score_bias, w_gate, w_up, w_down):
    T, D = x.shape
    F = w_gate.shape[2]

    logits = jnp.dot(x, router_w.T)
    scores = jax.nn.sigmoid(logits.astype(jnp.float32))
    scores_for_choice = scores + e_score_bias[None, :]
    _, topk_idx = jax.lax.top_k(scores_for_choice, _TOPK)
    topk_scores = jnp.take_along_axis(scores, topk_idx, axis=1)
    aff = topk_scores / (jnp.sum(topk_scores, axis=1, keepdims=True) + 1e-9)

    e_flat = topk_idx.reshape(-1).astype(jnp.int32)
    oh = (e_flat[:, None] == jnp.arange(_E, dtype=jnp.int32)[None, :]).astype(jnp.int32)
    incl = jnp.cumsum(oh, axis=0)
    ranks = jnp.take_along_axis(incl, e_flat[:, None], axis=1)[:, 0] - 1
    counts = incl[-1]
    padded = ((counts + _BLK - 1) // _BLK) * _BLK
    cum_padded = jnp.cumsum(padded)
    blk_off = cum_padded - padded
    pos = blk_off[e_flat] + ranks
    tok_of_asn = jnp.arange(T * _TOPK, dtype=jnp.int32) // _TOPK
    row_id = jnp.zeros((_PADT,), jnp.int32).at[pos].set(tok_of_asn)

    starts = jnp.arange(_NB, dtype=jnp.int32) * _BLK
    be_raw = jnp.sum((starts[:, None] >= cum_padded[None, :]).astype(jnp.int32), axis=1)
    valid = (starts < cum_padded[-1]).astype(jnp.int32)
    nvalid = jnp.sum(valid)
    last_e = be_raw[nvalid - 1]
    block_expert = jnp.where(valid == 1, be_raw, last_e).astype(jnp.int32)

    xs = jnp.take(x, row_id, axis=0)

    grid_spec = pltpu.PrefetchScalarGridSpec(
        num_scalar_prefetch=2,
        grid=(_NB,),
        in_specs=[
            pl.BlockSpec((_BLK, D), lambda b, be, bv: (b, 0)),
            pl.BlockSpec((1, D, F), lambda b, be, bv: (be[b], 0, 0)),
            pl.BlockSpec((1, D, F), lambda b, be, bv: (be[b], 0, 0)),
            pl.BlockSpec((1, F, D), lambda b, be, bv: (be[b], 0, 0)),
        ],
        out_specs=pl.BlockSpec((_BLK, D), lambda b, be, bv: (b, 0)),
    )
    ys = pl.pallas_call(
        _glu_body,
        grid_spec=grid_spec,
        out_shape=jax.ShapeDtypeStruct((_PADT, D), jnp.float32),
        compiler_params=pltpu.CompilerParams(
            dimension_semantics=("arbitrary",),
        ),
    )(block_expert, valid, xs, w_gate, w_up, w_down)

    pos2 = pos.reshape(T, _TOPK)
    out = (aff[:, 0:1] * jnp.take(ys, pos2[:, 0], axis=0)
           + aff[:, 1:2] * jnp.take(ys, pos2[:, 1], axis=0))
    return out.astype(x.dtype)


# speedup vs baseline: 1.5155x; 1.0033x over previous
import functools

import jax
import jax.numpy as jnp
from jax.experimental import pallas as pl
from jax.experimental.pallas import tpu as pltpu

_TOPK = 2
_E = 8
_BLK = 128
_NB = 4096 // _BLK + _E
_PADT = _NB * _BLK


def _glu_body(be_ref, bv_ref, xs_ref, wg_ref, wu_ref, wd_ref, ys_ref):
    b = pl.program_id(0)

    @pl.when(bv_ref[b] == 1)
    def _():
        xb = xs_ref[...]
        h = jnp.dot(xb, wg_ref[0], preferred_element_type=jnp.float32)
        u = jnp.dot(xb, wu_ref[0], preferred_element_type=jnp.float32)
        act = h * jax.lax.logistic(h) * u
        ys_ref[...] = jnp.dot(act, wd_ref[0], preferred_element_type=jnp.float32)

    @pl.when(bv_ref[b] == 0)
    def _():
        ys_ref[...] = jnp.zeros_like(ys_ref)


def kernel(x, router_w, e_score_bias, w_gate, w_up, w_down):
    T, D = x.shape
    F = w_gate.shape[2]

    logits = jnp.dot(x, router_w.T)
    scores = jax.nn.sigmoid(logits.astype(jnp.float32))
    scores_for_choice = scores + e_score_bias[None, :]
    _, topk_idx = jax.lax.top_k(scores_for_choice, _TOPK)
    topk_scores = jnp.take_along_axis(scores, topk_idx, axis=1)
    aff = topk_scores / (jnp.sum(topk_scores, axis=1, keepdims=True) + 1e-9)

    e_flat = topk_idx.reshape(-1).astype(jnp.int32)
    oh = (e_flat[:, None] == jnp.arange(_E, dtype=jnp.int32)[None, :]).astype(jnp.int32)
    incl = jnp.cumsum(oh, axis=0)
    ranks = jnp.take_along_axis(incl, e_flat[:, None], axis=1)[:, 0] - 1
    counts = incl[-1]
    padded = ((counts + _BLK - 1) // _BLK) * _BLK
    cum_padded = jnp.cumsum(padded)
    blk_off = cum_padded - padded
    pos = blk_off[e_flat] + ranks
    tok_of_asn = jnp.arange(T * _TOPK, dtype=jnp.int32) // _TOPK
    row_id = jnp.zeros((_PADT,), jnp.int32).at[pos].set(tok_of_asn)

    starts = jnp.arange(_NB, dtype=jnp.int32) * _BLK
    be_raw = jnp.sum((starts[:, None] >= cum_padded[None, :]).astype(jnp.int32), axis=1)
    valid = (starts < cum_padded[-1]).astype(jnp.int32)
    nvalid = jnp.sum(valid)
    last_e = be_raw[nvalid - 1]
    block_expert = jnp.where(valid == 1, be_raw, last_e).astype(jnp.int32)

    xs = jnp.take(x, row_id, axis=0)

    grid_spec = pltpu.PrefetchScalarGridSpec(
        num_scalar_prefetch=2,
        grid=(_NB,),
        in_specs=[
            pl.BlockSpec((_BLK, D), lambda b, be, bv: (b, 0)),
            pl.BlockSpec((1, D, F), lambda b, be, bv: (be[b], 0, 0)),
            pl.BlockSpec((1, D, F), lambda b, be, bv: (be[b], 0, 0)),
            pl.BlockSpec((1, F, D), lambda b, be, bv: (be[b], 0, 0)),
        ],
        out_specs=pl.BlockSpec((_BLK, D), lambda b, be, bv: (b, 0)),
    )
    ys = pl.pallas_call(
        _glu_body,
        grid_spec=grid_spec,
        out_shape=jax.ShapeDtypeStruct((_PADT, D), jnp.float32),
        compiler_params=pltpu.CompilerParams(
            dimension_semantics=("arbitrary",),
        ),
    )(block_expert, valid, xs, w_gate, w_up, w_down)

    pos2 = pos.reshape(T, _TOPK)
    out = (aff[:, 0:1] * jnp.take(ys, pos2[:, 0], axis=0)
           + aff[:, 1:2] * jnp.take(ys, pos2[:, 1], axis=0))
    return out.astype(x.dtype)
